# Initial kernel scaffold; baseline (speedup 1.0000x reference)
import jax
import jax.numpy as jnp
from jax.experimental import pallas as pl


def kernel(p1, p2, w1, b1, w2, b2):
    raise NotImplementedError("write your pallas kernel here")


# speedup vs baseline: 4.6279x; 4.6279x over previous
import jax
import jax.numpy as jnp
from jax.experimental import pallas as pl
from jax.experimental.pallas import tpu as pltpu

_B, _C, _DHW, _S = 8, 16, 64, 8
_POOL = _DHW // _S
_HALF = _B * _S ** 3
_N = 2 * _HALF
_INV_T = 10.0
_EPS = 1e-12

_DBLK = 16
_ND = _DBLK // _POOL
_R = 512
_NB = _HALF // _R


def _pool_body(x1_ref, x2_ref, o1_ref, o2_ref):
    wi = jax.lax.broadcasted_iota(jnp.int32, (_DHW, _S), 0)
    wo = jax.lax.broadcasted_iota(jnp.int32, (_DHW, _S), 1)
    pm = jnp.where(wi // _POOL == wo, 1.0 / (_POOL ** 3), 0.0).astype(jnp.float32)

    for x_ref, o_ref in ((x1_ref, o1_ref), (x2_ref, o2_ref)):
        x = x_ref[0]
        xs = x.reshape(_C, _ND, _POOL, _DHW, _DHW).sum(axis=2)
        xh = xs.reshape(_C, _ND, _S, _POOL, _DHW).sum(axis=3)
        y = jnp.dot(xh.reshape(_C * _ND * _S, _DHW), pm,
                    preferred_element_type=jnp.float32,
                    precision=jax.lax.Precision.HIGHEST)
        o_ref[0] = y.reshape(_C, _ND, _S, _S)


def _project_body(x_ref, w1_ref, b1_ref, w2_ref, b2_ref, o_ref):
    x = x_ref[...]
    h = jnp.dot(w1_ref[...], x, preferred_element_type=jnp.float32,
                precision=jax.lax.Precision.HIGHEST) + b1_ref[...]
    h = jnp.maximum(h, 0.0)
    f = jnp.dot(w2_ref[...], h, preferred_element_type=jnp.float32,
                precision=jax.lax.Precision.HIGHEST) + b2_ref[...]
    nrm = jnp.sqrt(jnp.sum(f * f, axis=0, keepdims=True))
    o_ref[...] = f / jnp.maximum(nrm, _EPS)


def _infonce_body(fr_ref, fp_ref, ff_ref, o_ref):
    j = pl.program_id(1)
    fr = fr_ref[...]
    frb = fr.astype(jnp.bfloat16)

    n_chunk = 4
    cw = _N // n_chunk
    s_all = jnp.zeros((_R, 1), dtype=jnp.float32)
    for q in range(n_chunk):
        fb = ff_ref[:, q * cw:(q + 1) * cw].astype(jnp.bfloat16)
        lg = jax.lax.dot_general(frb, fb, (((0,), (0,)), ((), ())),
                                 preferred_element_type=jnp.float32)
        e = jnp.exp(lg * _INV_T - _INV_T)
        s_all = s_all + jnp.sum(e, axis=1, keepdims=True)

    sq = frb.astype(jnp.float32)
    ones_c = jnp.ones((_C, 1), dtype=jnp.float32)
    dg = jax.lax.dot_general(sq * sq, ones_c, (((0,), (0,)), ((), ())),
                             preferred_element_type=jnp.float32,
                             precision=jax.lax.Precision.HIGHEST)
    s_off = s_all - jnp.exp(dg * _INV_T - _INV_T)

    lse_sum = jnp.sum(jnp.log(s_off)) + _INV_T * _R
    pos_sum = _INV_T * jnp.sum(fr * fp_ref[...])
    partial = lse_sum - pos_sum

    @pl.when(j == 0)
    def _():
        o_ref[0, 0, 0] = partial

    @pl.when(j > 0)
    def _():
        o_ref[0, 0, 0] = o_ref[0, 0, 0] + partial


def kernel(p1, p2, w1, b1, w2, b2):
    pooled = pl.pallas_call(
        _pool_body,
        grid=(_B, _DHW // _DBLK),
        in_specs=[
            pl.BlockSpec((1, _C, _DBLK, _DHW, _DHW), lambda b, d: (b, 0, d, 0, 0)),
            pl.BlockSpec((1, _C, _DBLK, _DHW, _DHW), lambda b, d: (b, 0, d, 0, 0)),
        ],
        out_specs=[
            pl.BlockSpec((1, _C, _ND, _S, _S), lambda b, d: (b, 0, d, 0, 0)),
            pl.BlockSpec((1, _C, _ND, _S, _S), lambda b, d: (b, 0, d, 0, 0)),
        ],
        out_shape=[
            jax.ShapeDtypeStruct((_B, _C, _S, _S, _S), jnp.float32),
            jax.ShapeDtypeStruct((_B, _C, _S, _S, _S), jnp.float32),
        ],
        compiler_params=pltpu.CompilerParams(
            dimension_semantics=("parallel", "arbitrary"),
        ),
        name="region_pool",
    )(p1, p2)

    mmT = jnp.concatenate(
        [p.reshape(_B, _C, _S ** 3).transpose(1, 0, 2).reshape(_C, _HALF)
         for p in pooled], axis=1)

    fT = pl.pallas_call(
        _project_body,
        grid=(2,),
        in_specs=[
            pl.BlockSpec((_C, _HALF), lambda i: (0, i)),
            pl.BlockSpec((_C, _C), lambda i: (0, 0)),
            pl.BlockSpec((_C, 1), lambda i: (0, 0)),
            pl.BlockSpec((_C, _C), lambda i: (0, 0)),
            pl.BlockSpec((_C, 1), lambda i: (0, 0)),
        ],
        out_specs=pl.BlockSpec((_C, _HALF), lambda i: (0, i)),
        out_shape=jax.ShapeDtypeStruct((_C, _N), jnp.float32),
        compiler_params=pltpu.CompilerParams(
            dimension_semantics=("parallel",),
        ),
        name="project_head",
    )(mmT, w1, b1.reshape(_C, 1), w2, b2.reshape(_C, 1))

    partials = pl.pallas_call(
        _infonce_body,
        grid=(2, _NB),
        in_specs=[
            pl.BlockSpec((_C, _R), lambda i, j: (0, i * _NB + j)),
            pl.BlockSpec((_C, _R), lambda i, j: (0, (1 - i) * _NB + j)),
            pl.BlockSpec((_C, _N), lambda i, j: (0, 0)),
        ],
        out_specs=pl.BlockSpec((1, 1, 1), lambda i, j: (i, 0, 0),
                               memory_space=pltpu.SMEM),
        out_shape=jax.ShapeDtypeStruct((2, 1, 1), jnp.float32),
        compiler_params=pltpu.CompilerParams(
            dimension_semantics=("parallel", "arbitrary"),
            vmem_limit_bytes=48 * 1024 * 1024,
        ),
        name="infonce_loss",
    )(fT, fT, fT)

    return (partials[0, 0, 0] + partials[1, 0, 0]) / _N


# speedup vs baseline: 4.7948x; 1.0361x over previous
import jax
import jax.numpy as jnp
from jax.experimental import pallas as pl
from jax.experimental.pallas import tpu as pltpu

_B, _C, _DHW, _S = 8, 16, 64, 8
_POOL = _DHW // _S
_SLAB = _S ** 3
_HALF = _B * _SLAB
_N = 2 * _HALF
_INV_T = 10.0
_EPS = 1e-12

_DBLK = 16
_ND = _DBLK // _POOL
_R = _SLAB
_NB = _HALF // _R
_NCHUNK = 4


def _pool_body(x1_ref, x2_ref, o_ref):
    wi = jax.lax.broadcasted_iota(jnp.int32, (_DHW, _S), 0)
    wo = jax.lax.broadcasted_iota(jnp.int32, (_DHW, _S), 1)
    pm = jnp.where(wi // _POOL == wo, 1.0 / (_POOL ** 3), 0.0).astype(jnp.float32)

    for v, x_ref in ((0, x1_ref), (1, x2_ref)):
        x = x_ref[0]
        hs = []
        for k in range(_ND):
            acc = x[:, _POOL * k]
            for d in range(1, _POOL):
                acc = acc + x[:, _POOL * k + d]
            hs.append(acc.reshape(_C, _S, _POOL, _DHW).sum(axis=2))
        xh = jnp.stack(hs, axis=1)
        y = jnp.dot(xh.reshape(_C * _ND * _S, _DHW), pm,
                    preferred_element_type=jnp.float32,
                    precision=jax.lax.Precision.HIGHEST)
        o_ref[v, 0] = y.reshape(_C, _ND, _S, _S)


def _loss_body(pf_ref, w1_ref, b1_ref, w2_ref, b2_ref, o_ref, fmat_ref):
    i = pl.program_id(0)
    j = pl.program_id(1)

    def _mlp(x):
        h = jnp.dot(w1_ref[...], x, preferred_element_type=jnp.float32,
                    precision=jax.lax.Precision.HIGHEST) + b1_ref[...]
        h = jnp.maximum(h, 0.0)
        f = jnp.dot(w2_ref[...], h, preferred_element_type=jnp.float32,
                    precision=jax.lax.Precision.HIGHEST) + b2_ref[...]
        nrm = jnp.sqrt(jnp.sum(f * f, axis=0, keepdims=True))
        return f / jnp.maximum(nrm, _EPS)

    @pl.when(j == 0)
    def _():
        for v in range(2):
            for b in range(_B):
                s = v * _B + b
                fmat_ref[:, s * _SLAB:(s + 1) * _SLAB] = (
                    _mlp(pf_ref[v, b]).astype(jnp.bfloat16))

    fr = _mlp(pf_ref[i, j])
    fp = _mlp(pf_ref[1 - i, j])
    frb = (fr * _INV_T).astype(jnp.bfloat16)

    cw = _N // _NCHUNK
    e_sum = jnp.zeros((_R, 1), dtype=jnp.float32)
    for q in range(_NCHUNK):
        fbq = fmat_ref[:, q * cw:(q + 1) * cw]
        lg = jax.lax.dot_general(frb, fbq, (((0,), (0,)), ((), ())),
                                 preferred_element_type=jnp.float32)
        e_sum = e_sum + jnp.sum(jnp.exp(lg), axis=1, keepdims=True)

    a = frb.astype(jnp.float32)
    bt = fr.astype(jnp.bfloat16).astype(jnp.float32)
    ones_c = jnp.ones((_C, 1), dtype=jnp.float32)
    dg = jax.lax.dot_general(a * bt, ones_c, (((0,), (0,)), ((), ())),
                             preferred_element_type=jnp.float32,
                             precision=jax.lax.Precision.HIGHEST)
    s_off = e_sum - jnp.exp(dg)

    partial = jnp.sum(jnp.log(s_off)) - _INV_T * jnp.sum(fr * fp)

    @pl.when(j == 0)
    def _():
        o_ref[0, 0, 0] = partial

    @pl.when(j > 0)
    def _():
        o_ref[0, 0, 0] = o_ref[0, 0, 0] + partial


def kernel(p1, p2, w1, b1, w2, b2):
    pooled = pl.pallas_call(
        _pool_body,
        grid=(_B, _DHW // _DBLK),
        in_specs=[
            pl.BlockSpec((1, _C, _DBLK, _DHW, _DHW), lambda b, d: (b, 0, d, 0, 0)),
            pl.BlockSpec((1, _C, _DBLK, _DHW, _DHW), lambda b, d: (b, 0, d, 0, 0)),
        ],
        out_specs=pl.BlockSpec((2, 1, _C, _ND, _S, _S),
                               lambda b, d: (0, b, 0, d, 0, 0)),
        out_shape=jax.ShapeDtypeStruct((2, _B, _C, _S, _S, _S), jnp.float32),
        compiler_params=pltpu.CompilerParams(
            dimension_semantics=("parallel", "arbitrary"),
        ),
        name="region_pool",
    )(p1, p2)

    pooled = pooled.reshape(2, _B, _C, _SLAB)

    partials = pl.pallas_call(
        _loss_body,
        grid=(2, _NB),
        in_specs=[
            pl.BlockSpec((2, _B, _C, _SLAB), lambda i, j: (0, 0, 0, 0)),
            pl.BlockSpec((_C, _C), lambda i, j: (0, 0)),
            pl.BlockSpec((_C, 1), lambda i, j: (0, 0)),
            pl.BlockSpec((_C, _C), lambda i, j: (0, 0)),
            pl.BlockSpec((_C, 1), lambda i, j: (0, 0)),
        ],
        out_specs=pl.BlockSpec((1, 1, 1), lambda i, j: (i, 0, 0),
                               memory_space=pltpu.SMEM),
        out_shape=jax.ShapeDtypeStruct((2, 1, 1), jnp.float32),
        scratch_shapes=[pltpu.VMEM((_C, _N), jnp.bfloat16)],
        compiler_params=pltpu.CompilerParams(
            dimension_semantics=("parallel", "arbitrary"),
            vmem_limit_bytes=48 * 1024 * 1024,
        ),
        name="head_infonce_loss",
    )(pooled, w1, b1.reshape(_C, 1), w2, b2.reshape(_C, 1))

    return (partials[0, 0, 0] + partials[1, 0, 0]) / _N


# speedup vs baseline: 4.8084x; 1.0028x over previous
import jax
import jax.numpy as jnp
from jax.experimental import pallas as pl
from jax.experimental.pallas import tpu as pltpu

_B, _C, _DHW, _S = 8, 16, 64, 8
_POOL = _DHW // _S
_SLAB = _S ** 3
_HALF = _B * _SLAB
_N = 2 * _HALF
_INV_T = 10.0
_SC2 = 14.426950408889634
_EPS = 1e-12

_DBLK = 16
_ND = _DBLK // _POOL
_R = _SLAB
_NB = _HALF // _R
_NCHUNK = 4


def _pool_body(x1_ref, x2_ref, o_ref):
    wi = jax.lax.broadcasted_iota(jnp.int32, (_DHW, _S), 0)
    wo = jax.lax.broadcasted_iota(jnp.int32, (_DHW, _S), 1)
    pm = jnp.where(wi // _POOL == wo, 1.0 / (_POOL ** 3), 0.0).astype(jnp.float32)

    for v, x_ref in ((0, x1_ref), (1, x2_ref)):
        x = x_ref[0]
        hs = []
        for k in range(_ND):
            acc = x[:, _POOL * k]
            for d in range(1, _POOL):
                acc = acc + x[:, _POOL * k + d]
            hs.append(acc.reshape(_C, _S, _POOL, _DHW).sum(axis=2))
        xh = jnp.stack(hs, axis=1)
        y = jnp.dot(xh.reshape(_C * _ND * _S, _DHW), pm,
                    preferred_element_type=jnp.float32,
                    precision=jax.lax.Precision.HIGHEST)
        o_ref[v, 0] = y.reshape(_C, _ND, _S, _S)


def _loss_body(pf_ref, w1_ref, b1_ref, w2_ref, b2_ref, o_ref, fmat_ref):
    i = pl.program_id(0)
    j = pl.program_id(1)

    def _mlp(x):
        h = jnp.dot(w1_ref[...], x, preferred_element_type=jnp.float32,
                    precision=jax.lax.Precision.HIGHEST) + b1_ref[...]
        h = jnp.maximum(h, 0.0)
        f = jnp.dot(w2_ref[...], h, preferred_element_type=jnp.float32,
                    precision=jax.lax.Precision.HIGHEST) + b2_ref[...]
        nrm = jnp.sqrt(jnp.sum(f * f, axis=0, keepdims=True))
        return f / jnp.maximum(nrm, _EPS)

    @pl.when(j == 0)
    def _():
        for v in range(2):
            for b in range(_B):
                s = v * _B + b
                fmat_ref[:, s * _SLAB:(s + 1) * _SLAB] = (
                    _mlp(pf_ref[v, b]).astype(jnp.bfloat16))

    fr = _mlp(pf_ref[i, j])
    fp = _mlp(pf_ref[1 - i, j])
    frb = (fr * _SC2).astype(jnp.bfloat16)

    cw = _N // _NCHUNK
    e_sum = jnp.zeros((_R, 1), dtype=jnp.float32)
    for q in range(_NCHUNK):
        fbq = fmat_ref[:, q * cw:(q + 1) * cw]
        lg = jax.lax.dot_general(frb, fbq, (((0,), (0,)), ((), ())),
                                 preferred_element_type=jnp.float32)
        e_sum = e_sum + jnp.sum(jnp.exp2(lg), axis=1, keepdims=True)

    a = frb.astype(jnp.float32)
    bt = fr.astype(jnp.bfloat16).astype(jnp.float32)
    ones_c = jnp.ones((_C, 1), dtype=jnp.float32)
    dg = jax.lax.dot_general(a * bt, ones_c, (((0,), (0,)), ((), ())),
                             preferred_element_type=jnp.float32,
                             precision=jax.lax.Precision.HIGHEST)
    s_off = e_sum - jnp.exp2(dg)

    partial = jnp.sum(jnp.log(s_off)) - _INV_T * jnp.sum(fr * fp)

    @pl.when(j == 0)
    def _():
        o_ref[0, 0, 0] = partial

    @pl.when(j > 0)
    def _():
        o_ref[0, 0, 0] = o_ref[0, 0, 0] + partial


def kernel(p1, p2, w1, b1, w2, b2):
    pooled = pl.pallas_call(
        _pool_body,
        grid=(_B, _DHW // _DBLK),
        in_specs=[
            pl.BlockSpec((1, _C, _DBLK, _DHW, _DHW), lambda b, d: (b, 0, d, 0, 0)),
            pl.BlockSpec((1, _C, _DBLK, _DHW, _DHW), lambda b, d: (b, 0, d, 0, 0)),
        ],
        out_specs=pl.BlockSpec((2, 1, _C, _ND, _S, _S),
                               lambda b, d: (0, b, 0, d, 0, 0)),
        out_shape=jax.ShapeDtypeStruct((2, _B, _C, _S, _S, _S), jnp.float32),
        compiler_params=pltpu.CompilerParams(
            dimension_semantics=("arbitrary", "arbitrary"),
        ),
        name="region_pool",
    )(p1, p2)

    pooled = pooled.reshape(2, _B, _C, _SLAB)

    partials = pl.pallas_call(
        _loss_body,
        grid=(2, _NB),
        in_specs=[
            pl.BlockSpec((2, _B, _C, _SLAB), lambda i, j: (0, 0, 0, 0)),
            pl.BlockSpec((_C, _C), lambda i, j: (0, 0)),
            pl.BlockSpec((_C, 1), lambda i, j: (0, 0)),
            pl.BlockSpec((_C, _C), lambda i, j: (0, 0)),
            pl.BlockSpec((_C, 1), lambda i, j: (0, 0)),
        ],
        out_specs=pl.BlockSpec((1, 1, 1), lambda i, j: (i, 0, 0),
                               memory_space=pltpu.SMEM),
        out_shape=jax.ShapeDtypeStruct((2, 1, 1), jnp.float32),
        scratch_shapes=[pltpu.VMEM((_C, _N), jnp.bfloat16)],
        compiler_params=pltpu.CompilerParams(
            dimension_semantics=("arbitrary", "arbitrary"),
            vmem_limit_bytes=48 * 1024 * 1024,
        ),
        name="head_infonce_loss",
    )(pooled, w1, b1.reshape(_C, 1), w2, b2.reshape(_C, 1))

    return (partials[0, 0, 0] + partials[1, 0, 0]) / _N
